# baseline (speedup 1.0000x reference)
import functools

import jax
import jax.numpy as jnp
from jax import lax
from jax.experimental import pallas as pl
from jax.experimental.pallas import tpu as pltpu
from jax.experimental.pallas import tpu_sc as plsc

_B, _S, _D = 1024, 200, 64
_ROWS = _B * _S
_GROUP = 128
_NGRP = _ROWS // _GROUP
_BBLK = 64
_GRID = _B // _BBLK


@functools.lru_cache(maxsize=1)
def _sc_gather_fn():
    info = plsc.get_sparse_core_info()
    nw = info.num_cores * info.num_subcores
    gpw = _NGRP // nw
    mesh = plsc.VectorSubcoreMesh(core_axis_name="c", subcore_axis_name="s")

    @functools.partial(
        pl.kernel,
        mesh=mesh,
        out_type=jax.ShapeDtypeStruct((_ROWS, _D), jnp.bfloat16),
        scratch_types=[
            pltpu.VMEM((gpw, _GROUP), jnp.int32),
            pltpu.VMEM((_GROUP, _D), jnp.bfloat16),
            pltpu.SemaphoreType.DMA,
        ],
        compiler_params=pltpu.CompilerParams(use_tc_tiling_on_sc=False),
    )
    def gather_k(table, idx_hbm, out_hbm, idx_v, rows_v, sem):
        wid = lax.axis_index("s") * info.num_cores + lax.axis_index("c")
        pltpu.sync_copy(idx_hbm.at[wid], idx_v)
        base = wid * gpw * _GROUP

        def body(j, carry):
            pltpu.async_copy(table.at[idx_v.at[j]], rows_v, sem).wait()
            pltpu.sync_copy(rows_v, out_hbm.at[pl.ds(base + j * _GROUP, _GROUP)])
            return carry

        lax.fori_loop(0, gpw, body, 0)

    def run(table, idx2d):
        return gather_k(table, idx2d.reshape(nw, gpw, _GROUP))

    return run


def _tc_body(g_ref, w48_ref, b1_ref, w2big_ref, w2m_ref, b2t_ref, dt_ref,
             gsel_ref, out_ref):
    i = pl.program_id(0)
    g = g_ref[...]
    q = jnp.dot(g, w48_ref[...], preferred_element_type=jnp.float32)
    q3 = q.reshape(_BBLK, _S, 48)
    qt = lax.transpose(q3, (0, 2, 1))
    zpad = jnp.zeros((_BBLK, 48, 6), jnp.float32)
    qtp = jnp.concatenate([zpad, qt, zpad], axis=2)
    tot = None
    for k in range(7):
        sl = lax.slice(qtp, (0, 6 * k, k), (_BBLK, 6 * k + 6, k + 206))
        tot = sl if tot is None else tot + sl
    conv1t = tot + b1_ref[...][None]

    exc = jnp.mean(conv1t, axis=1)
    iota = lax.broadcasted_iota(jnp.int32, (_BBLK, 206), 1)
    big = jnp.int32(1 << 30)
    picks = []
    ew = exc
    for _ in range(8):
        m = jnp.max(ew, axis=1, keepdims=True)
        pos = jnp.min(jnp.where(ew == m, iota, big), axis=1, keepdims=True)
        sel = iota == pos
        self32 = jnp.where(sel, 1.0, 0.0)
        picks.append(jnp.sum(self32[:, None, :] * conv1t, axis=2))
        ew = jnp.where(sel, -jnp.inf, ew)
    s1 = jnp.concatenate(picks, axis=1)
    s1 = 1.0 / (1.0 + jnp.exp(-s1))

    conv2 = jnp.dot(s1, w2big_ref[...],
                    preferred_element_type=jnp.float32) + b2t_ref[...]
    exc2 = jnp.dot(s1, w2m_ref[...],
                   preferred_element_type=jnp.float32)
    iota2 = lax.broadcasted_iota(jnp.int32, (_BBLK, 12), 1)
    keep = jnp.zeros((_BBLK, 12), jnp.float32)
    ew2 = exc2
    for _ in range(4):
        m = jnp.max(ew2, axis=1, keepdims=True)
        pos = jnp.min(jnp.where(ew2 == m, iota2, big), axis=1, keepdims=True)
        sel = iota2 == pos
        keep = keep + jnp.where(sel, 1.0, 0.0)
        ew2 = jnp.where(sel, -jnp.inf, ew2)
    z = conv2 * dt_ref[...]
    zg = jnp.dot(z, gsel_ref[...],
                 preferred_element_type=jnp.float32)
    part = (0.25 * jnp.sum(keep * zg)).reshape(1, 1)

    @pl.when(i == 0)
    def _init():
        out_ref[...] = jnp.zeros((1, 1), jnp.float32)

    out_ref[...] += part


def _post_gather(g2d, conv1_w, conv1_b, conv2_w, conv2_b, dense_w):
    f32 = jnp.float32
    w1r = conv1_w.transpose(1, 0, 2).reshape(_D, 42)
    w48 = jnp.concatenate([w1r, jnp.zeros((_D, 6), f32)], axis=1)
    w48 = w48.astype(jnp.bfloat16)
    b1c = conv1_b.reshape(6, 1)
    jj = jnp.arange(8)[:, None]
    tt = jnp.arange(12)[None, :]
    kk = jj - tt + 4
    valid = (kk >= 0) & (kk < 5)
    w2k = conv2_w[jnp.clip(kk, 0, 4)]
    w2k = jnp.where(valid[:, :, None, None], w2k, 0.0)
    w2big = w2k.transpose(0, 2, 1, 3).reshape(48, 168)
    w2m = w2big.reshape(48, 12, 14).mean(axis=2)
    b2t = jnp.tile(conv2_b, 12).reshape(1, 168)
    dt = jnp.tile(dense_w.reshape(-1), 12).reshape(1, 168)
    gsel = jnp.kron(jnp.eye(12, dtype=f32), jnp.ones((14, 1), f32))
    acc = pl.pallas_call(
        _tc_body,
        grid=(_GRID,),
        in_specs=[
            pl.BlockSpec((_BBLK * _S, _D), lambda i: (i, 0)),
            pl.BlockSpec((_D, 48), lambda i: (0, 0)),
            pl.BlockSpec((6, 1), lambda i: (0, 0)),
            pl.BlockSpec((48, 168), lambda i: (0, 0)),
            pl.BlockSpec((48, 12), lambda i: (0, 0)),
            pl.BlockSpec((1, 168), lambda i: (0, 0)),
            pl.BlockSpec((1, 168), lambda i: (0, 0)),
            pl.BlockSpec((168, 12), lambda i: (0, 0)),
        ],
        out_specs=pl.BlockSpec((1, 1), lambda i: (0, 0)),
        out_shape=jax.ShapeDtypeStruct((1, 1), jnp.float32),
        compiler_params=pltpu.CompilerParams(
            dimension_semantics=("arbitrary",)),
    )(g2d, w48, b1c, w2big, w2m, b2t, dt, gsel)
    return acc


def kernel(x, embeddings, conv1_w, conv1_b, conv2_w, conv2_b, dense_w, dense_b):
    emb16 = embeddings.astype(jnp.bfloat16)
    idx = x.astype(jnp.int32).reshape(_NGRP, _GROUP)
    gathered = _sc_gather_fn()(emb16, idx)
    acc = _post_gather(gathered, conv1_w, conv1_b, conv2_w, conv2_b, dense_w)
    return jax.nn.sigmoid(acc[0, 0] / _B + dense_b[0])


# speedup vs baseline: 1.2979x; 1.2979x over previous
import functools

import jax
import jax.numpy as jnp
from jax import lax
from jax.experimental import pallas as pl
from jax.experimental.pallas import tpu as pltpu
from jax.experimental.pallas import tpu_sc as plsc

_B, _S, _D = 1024, 200, 64
_ROWS = _B * _S
_GROUP = 128
_NGRP = _ROWS // _GROUP
_BBLK = 64
_GRID = _B // _BBLK
_PAIRW = 2 * _D


@functools.lru_cache(maxsize=1)
def _sc_gather_fn():
    info = plsc.get_sparse_core_info()
    nw = info.num_cores * info.num_subcores
    gpw = _NGRP // nw
    mesh = plsc.VectorSubcoreMesh(core_axis_name="c", subcore_axis_name="s")

    @functools.partial(
        pl.kernel,
        mesh=mesh,
        out_type=jax.ShapeDtypeStruct((_ROWS, _PAIRW), jnp.float32),
        scratch_types=[
            pltpu.VMEM((gpw, _GROUP), jnp.int32),
            pltpu.VMEM((_GROUP, _PAIRW), jnp.float32),
            pltpu.SemaphoreType.DMA,
        ],
    )
    def gather_k(table, idx_hbm, out_hbm, idx_v, rows_v, sem):
        wid = lax.axis_index("s") * info.num_cores + lax.axis_index("c")
        pltpu.sync_copy(idx_hbm.at[wid], idx_v)
        base = wid * gpw * _GROUP

        def body(j, carry):
            pltpu.async_copy(table.at[idx_v.at[j]], rows_v, sem).wait()
            pltpu.sync_copy(rows_v, out_hbm.at[pl.ds(base + j * _GROUP, _GROUP)])
            return carry

        lax.fori_loop(0, gpw, body, 0)

    def run(table, idx2d):
        return gather_k(table, idx2d.reshape(nw, gpw, _GROUP))

    return run


_VB = 16384
_TGRID = -(-1000000 // _VB)


def _transpose_body(xt_ref, out_ref):
    blk = xt_ref[...]
    out_ref[...] = lax.transpose(blk, (1, 0))


def _make_table(embeddings):
    embT = embeddings.T
    rows = pl.pallas_call(
        _transpose_body,
        grid=(_TGRID,),
        in_specs=[pl.BlockSpec((_D, _VB), lambda i: (0, i))],
        out_specs=pl.BlockSpec((_VB, _D), lambda i: (i, 0)),
        out_shape=jax.ShapeDtypeStruct((1000000, _D), jnp.float32),
    )(embT)
    return rows.reshape(500000, _PAIRW)


def _tc_body(g_ref, par_ref, w96_ref, b1_ref, w2big_ref, w2m_ref, b2t_ref,
             dt_ref, gsel_ref, out_ref):
    i = pl.program_id(0)
    g = g_ref[...]
    q = jnp.dot(g, w96_ref[...], preferred_element_type=jnp.float32)
    q3 = q.reshape(_BBLK, _S, 96)
    qt = lax.transpose(q3, (0, 2, 1))
    par = par_ref[...]
    par3 = par[:, None, :]
    qtop = lax.slice(qt, (0, 0, 0), (_BBLK, 48, _S))
    qbot = lax.slice(qt, (0, 48, 0), (_BBLK, 96, _S))
    qsel = qtop + (qbot - qtop) * par3
    zpad = jnp.zeros((_BBLK, 48, 6), jnp.float32)
    qtp = jnp.concatenate([zpad, qsel, zpad], axis=2)
    tot = None
    for k in range(7):
        sl = lax.slice(qtp, (0, 6 * k, k), (_BBLK, 6 * k + 6, k + 206))
        tot = sl if tot is None else tot + sl
    conv1t = tot + b1_ref[...][None]

    exc = jnp.mean(conv1t, axis=1)
    iota = lax.broadcasted_iota(jnp.int32, (_BBLK, 206), 1)
    big = jnp.int32(1 << 30)
    picks = []
    ew = exc
    for _ in range(8):
        m = jnp.max(ew, axis=1, keepdims=True)
        pos = jnp.min(jnp.where(ew == m, iota, big), axis=1, keepdims=True)
        sel = iota == pos
        self32 = jnp.where(sel, 1.0, 0.0)
        picks.append(jnp.sum(self32[:, None, :] * conv1t, axis=2))
        ew = jnp.where(sel, -jnp.inf, ew)
    s1 = jnp.concatenate(picks, axis=1)
    s1 = 1.0 / (1.0 + jnp.exp(-s1))

    conv2 = jnp.dot(s1, w2big_ref[...],
                    preferred_element_type=jnp.float32) + b2t_ref[...]
    exc2 = jnp.dot(s1, w2m_ref[...],
                   preferred_element_type=jnp.float32)
    iota2 = lax.broadcasted_iota(jnp.int32, (_BBLK, 12), 1)
    keep = jnp.zeros((_BBLK, 12), jnp.float32)
    ew2 = exc2
    for _ in range(4):
        m = jnp.max(ew2, axis=1, keepdims=True)
        pos = jnp.min(jnp.where(ew2 == m, iota2, big), axis=1, keepdims=True)
        sel = iota2 == pos
        keep = keep + jnp.where(sel, 1.0, 0.0)
        ew2 = jnp.where(sel, -jnp.inf, ew2)
    z = conv2 * dt_ref[...]
    zg = jnp.dot(z, gsel_ref[...],
                 preferred_element_type=jnp.float32)
    part = (0.25 * jnp.sum(keep * zg)).reshape(1, 1)

    @pl.when(i == 0)
    def _init():
        out_ref[...] = jnp.zeros((1, 1), jnp.float32)

    out_ref[...] += part


def _post_gather(g2d, parf, conv1_w, conv1_b, conv2_w, conv2_b, dense_w):
    f32 = jnp.float32
    w1r = conv1_w.transpose(1, 0, 2).reshape(_D, 42)
    w48 = jnp.concatenate([w1r, jnp.zeros((_D, 6), f32)], axis=1)
    w96 = jnp.zeros((_PAIRW, 96), f32)
    w96 = w96.at[:_D, :48].set(w48).at[_D:, 48:].set(w48)
    b1c = conv1_b.reshape(6, 1)
    jj = jnp.arange(8)[:, None]
    tt = jnp.arange(12)[None, :]
    kk = jj - tt + 4
    valid = (kk >= 0) & (kk < 5)
    w2k = conv2_w[jnp.clip(kk, 0, 4)]
    w2k = jnp.where(valid[:, :, None, None], w2k, 0.0)
    w2big = w2k.transpose(0, 2, 1, 3).reshape(48, 168)
    w2m = w2big.reshape(48, 12, 14).mean(axis=2)
    b2t = jnp.tile(conv2_b, 12).reshape(1, 168)
    dt = jnp.tile(dense_w.reshape(-1), 12).reshape(1, 168)
    gsel = jnp.kron(jnp.eye(12, dtype=f32), jnp.ones((14, 1), f32))
    acc = pl.pallas_call(
        _tc_body,
        grid=(_GRID,),
        in_specs=[
            pl.BlockSpec((_BBLK * _S, _PAIRW), lambda i: (i, 0)),
            pl.BlockSpec((_BBLK, _S), lambda i: (i, 0)),
            pl.BlockSpec((_PAIRW, 96), lambda i: (0, 0)),
            pl.BlockSpec((6, 1), lambda i: (0, 0)),
            pl.BlockSpec((48, 168), lambda i: (0, 0)),
            pl.BlockSpec((48, 12), lambda i: (0, 0)),
            pl.BlockSpec((1, 168), lambda i: (0, 0)),
            pl.BlockSpec((1, 168), lambda i: (0, 0)),
            pl.BlockSpec((168, 12), lambda i: (0, 0)),
        ],
        out_specs=pl.BlockSpec((1, 1), lambda i: (0, 0)),
        out_shape=jax.ShapeDtypeStruct((1, 1), jnp.float32),
        compiler_params=pltpu.CompilerParams(
            dimension_semantics=("arbitrary",)),
    )(g2d, parf, w96, b1c, w2big, w2m, b2t, dt, gsel)
    return acc


def kernel(x, embeddings, conv1_w, conv1_b, conv2_w, conv2_b, dense_w, dense_b):
    xi = x.astype(jnp.int32)
    table = _make_table(embeddings)
    idx = (xi >> 1).reshape(_NGRP, _GROUP)
    parf = (xi & 1).astype(jnp.float32)
    gathered = _sc_gather_fn()(table, idx)
    acc = _post_gather(gathered, parf, conv1_w, conv1_b, conv2_w, conv2_b,
                       dense_w)
    return jax.nn.sigmoid(acc[0, 0] / _B + dense_b[0])
